# baseline (speedup 1.0000x reference)
import functools

import jax
import jax.numpy as jnp
from jax import lax
from jax.experimental import pallas as pl
from jax.experimental.pallas import tpu as pltpu
from jax.experimental.pallas import tpu_sc as plsc

_B = 16384
_D = 64
_V = 1000000
_NW = 32
_BPW = _B // _NW
_CH = 128
_NCH = _BPW // _CH

_CCH = 256
_RPW = 31744
_NFULL = _RPW // _CCH
_LAST_LO = 31 * _RPW
_LAST_FULL = (_V - _LAST_LO) // _CCH
_TAIL_LO = _LAST_LO + _LAST_FULL * _CCH
_TAIL = _V - _TAIL_LO


def _tc_concat(tug, tum, tig, tim):
    bs = 1000
    grid = (_V // bs,)
    half = pl.BlockSpec((bs, _D), lambda i: (i, 0))
    out_s = pl.BlockSpec((bs, 2 * _D), lambda i: (i, 0))

    def body(a_r, b_r, c_r, d_r, u_r, i_r):
        u_r[...] = jnp.concatenate([a_r[...], b_r[...]], axis=1)
        i_r[...] = jnp.concatenate([c_r[...], d_r[...]], axis=1)

    return pl.pallas_call(
        body,
        grid=grid,
        in_specs=[half, half, half, half],
        out_specs=[out_s, out_s],
        out_shape=[jax.ShapeDtypeStruct((_V, 2 * _D), jnp.float32)
                   for _ in range(2)],
    )(tug, tum, tig, tim)


def _sc_gather(user, item, U, I):
    mesh = plsc.VectorSubcoreMesh(core_axis_name="c", subcore_axis_name="s")
    out_t = [jax.ShapeDtypeStruct((_B, 2 * _D), jnp.float32) for _ in range(2)]
    scratch = (
        [pltpu.VMEM((_NCH, _CH), jnp.int32) for _ in range(2)]
        + [pltpu.VMEM((_CH, 2 * _D), jnp.float32) for _ in range(2)]
        + [pltpu.SemaphoreType.DMA for _ in range(4)]
    )

    @functools.partial(pl.kernel, mesh=mesh, out_type=out_t,
                       scratch_types=scratch)
    def body(user_h, item_h, U_h, I_h, o_gu, o_gi, idxu, idxi, buf0, buf1,
             gsem0, gsem1, wsem0, wsem1):
        c = lax.axis_index("c")
        s = lax.axis_index("s")
        base = (s * 2 + c) * _BPW

        for j in range(_NCH):
            pltpu.sync_copy(user_h.at[pl.ds(base + j * _CH, _CH)], idxu.at[j])
            pltpu.sync_copy(item_h.at[pl.ds(base + j * _CH, _CH)], idxi.at[j])

        bufs = [buf0, buf1]
        gsems = [gsem0, gsem1]
        wsems = [wsem0, wsem1]

        phases = [(U_h, idxu, o_gu, j) for j in range(_NCH)] \
               + [(I_h, idxi, o_gi, j) for j in range(_NCH)]

        def fire(p):
            tab, ix, _, j = phases[p]
            k = p % 2
            return pltpu.async_copy(tab.at[ix.at[j]], bufs[k], gsems[k])

        inflight = fire(0)
        write_handles = [None, None]
        for p in range(8):
            inflight.wait()
            nxt = fire(p + 1) if p < 7 else None
            k = p % 2
            if write_handles[k] is not None:
                write_handles[k].wait()
            _, _, out, j = phases[p]
            write_handles[k] = pltpu.async_copy(
                bufs[k], out.at[pl.ds(base + j * _CH, _CH)], wsems[k])
            inflight = nxt
        for wh in write_handles:
            wh.wait()

    return body(user, item, U, I)


def _tc_mlp(gu, gi, W1, b1, W2, b2, W3, b3, Wout, bout):
    bs = 2048
    grid = (_B // bs,)
    b1r = b1.reshape(1, -1)
    b2r = b2.reshape(1, -1)
    b3r = b3.reshape(1, -1)
    wa = Wout[:_D, 0].reshape(1, _D)
    wb = Wout[_D:, 0].reshape(1, -1)
    bor = bout.reshape(1, 1)

    def body(gu_r, gi_r, w1_r, b1_r, w2_r, b2_r, w3_r, b3_r,
             wa_r, wb_r, bo_r, out_r):
        h = jnp.dot(gu_r[:, _D:], w1_r[:_D, :],
                    preferred_element_type=jnp.float32)
        h = h + jnp.dot(gi_r[:, _D:], w1_r[_D:, :],
                        preferred_element_type=jnp.float32)
        h = jax.nn.relu(h + b1_r[...])
        h = jax.nn.relu(jnp.dot(h, w2_r[...],
                                preferred_element_type=jnp.float32) + b2_r[...])
        h = jax.nn.relu(jnp.dot(h, w3_r[...],
                                preferred_element_type=jnp.float32) + b3_r[...])
        g = gu_r[:, :_D] * gi_r[:, :_D]
        p = (jnp.sum(g * wa_r[...], axis=1, keepdims=True)
             + jnp.sum(h * wb_r[...], axis=1, keepdims=True) + bo_r[0, 0])
        out_r[...] = 1.0 / (1.0 + jnp.exp(-p))

    full = lambda a: pl.BlockSpec(a.shape, lambda i: (0,) * a.ndim)
    emb = pl.BlockSpec((bs, 2 * _D), lambda i: (i, 0))
    out = pl.pallas_call(
        body,
        grid=grid,
        in_specs=[emb, emb,
                  full(W1), full(b1r), full(W2), full(b2r),
                  full(W3), full(b3r), full(wa), full(wb), full(bor)],
        out_specs=pl.BlockSpec((bs, 1), lambda i: (i, 0)),
        out_shape=jax.ShapeDtypeStruct((_B, 1), jnp.float32),
    )(gu, gi, W1, b1r, W2, b2r, W3, b3r, wa, wb, bor)
    return out.reshape(_B)


def kernel(user, item, user_gmf_emb, item_gmf_emb, user_mlp_emb, item_mlp_emb,
           W1, b1, W2, b2, W3, b3, Wout, bout):
    U, I = _tc_concat(user_gmf_emb, user_mlp_emb, item_gmf_emb, item_mlp_emb)
    gu, gi = _sc_gather(user, item, U, I)
    return _tc_mlp(gu, gi, W1, b1, W2, b2, W3, b3, Wout, bout)


# speedup vs baseline: 1.1232x; 1.1232x over previous
import functools

import jax
import jax.numpy as jnp
from jax import lax
from jax.experimental import pallas as pl
from jax.experimental.pallas import tpu as pltpu
from jax.experimental.pallas import tpu_sc as plsc

_B = 16384
_D = 64
_V = 1000000
_NW = 32
_BPW = _B // _NW
_CH = 128
_NCH = _BPW // _CH

_CCH = 256
_RPW = 31744
_NFULL = _RPW // _CCH
_LAST_LO = 31 * _RPW
_LAST_FULL = (_V - _LAST_LO) // _CCH
_TAIL_LO = _LAST_LO + _LAST_FULL * _CCH
_TAIL = _V - _TAIL_LO


def _tc_concat(tug, tum, tig, tim):
    bs = 4000
    grid = (_V // bs,)
    half = pl.BlockSpec((bs, _D), lambda i: (i, 0))
    out_s = pl.BlockSpec((bs, 2 * _D), lambda i: (i, 0))

    def body(a_r, b_r, c_r, d_r, u_r, i_r):
        u_r[...] = jnp.concatenate([a_r[...], b_r[...]], axis=1)
        i_r[...] = jnp.concatenate([c_r[...], d_r[...]], axis=1)

    return pl.pallas_call(
        body,
        grid=grid,
        in_specs=[half, half, half, half],
        out_specs=[out_s, out_s],
        out_shape=[jax.ShapeDtypeStruct((_V, 2 * _D), jnp.float32)
                   for _ in range(2)],
    )(tug, tum, tig, tim)


def _sc_gather(user, item, U, I):
    mesh = plsc.VectorSubcoreMesh(core_axis_name="c", subcore_axis_name="s")
    out_t = [jax.ShapeDtypeStruct((_B, 2 * _D), jnp.float32) for _ in range(2)]
    scratch = (
        [pltpu.VMEM((_NCH, _CH), jnp.int32) for _ in range(2)]
        + [pltpu.VMEM((_CH, 2 * _D), jnp.float32) for _ in range(2)]
        + [pltpu.SemaphoreType.DMA for _ in range(4)]
    )

    @functools.partial(pl.kernel, mesh=mesh, out_type=out_t,
                       scratch_types=scratch)
    def body(user_h, item_h, U_h, I_h, o_gu, o_gi, idxu, idxi, buf0, buf1,
             gsem0, gsem1, wsem0, wsem1):
        c = lax.axis_index("c")
        s = lax.axis_index("s")
        base = (s * 2 + c) * _BPW

        for j in range(_NCH):
            pltpu.sync_copy(user_h.at[pl.ds(base + j * _CH, _CH)], idxu.at[j])
            pltpu.sync_copy(item_h.at[pl.ds(base + j * _CH, _CH)], idxi.at[j])

        bufs = [buf0, buf1]
        gsems = [gsem0, gsem1]
        wsems = [wsem0, wsem1]

        phases = [(U_h, idxu, o_gu, j) for j in range(_NCH)] \
               + [(I_h, idxi, o_gi, j) for j in range(_NCH)]

        def fire(p):
            tab, ix, _, j = phases[p]
            k = p % 2
            return pltpu.async_copy(tab.at[ix.at[j]], bufs[k], gsems[k])

        inflight = fire(0)
        write_handles = [None, None]
        for p in range(8):
            inflight.wait()
            nxt = fire(p + 1) if p < 7 else None
            k = p % 2
            if write_handles[k] is not None:
                write_handles[k].wait()
            _, _, out, j = phases[p]
            write_handles[k] = pltpu.async_copy(
                bufs[k], out.at[pl.ds(base + j * _CH, _CH)], wsems[k])
            inflight = nxt
        for wh in write_handles:
            wh.wait()

    return body(user, item, U, I)


def _tc_mlp(gu, gi, W1, b1, W2, b2, W3, b3, Wout, bout):
    bs = 2048
    grid = (_B // bs,)
    b1r = b1.reshape(1, -1)
    b2r = b2.reshape(1, -1)
    b3r = b3.reshape(1, -1)
    wa = Wout[:_D, 0].reshape(1, _D)
    wb = Wout[_D:, 0].reshape(1, -1)
    bor = bout.reshape(1, 1)

    def body(gu_r, gi_r, w1_r, b1_r, w2_r, b2_r, w3_r, b3_r,
             wa_r, wb_r, bo_r, out_r):
        h = jnp.dot(gu_r[:, _D:], w1_r[:_D, :],
                    preferred_element_type=jnp.float32)
        h = h + jnp.dot(gi_r[:, _D:], w1_r[_D:, :],
                        preferred_element_type=jnp.float32)
        h = jax.nn.relu(h + b1_r[...])
        h = jax.nn.relu(jnp.dot(h, w2_r[...],
                                preferred_element_type=jnp.float32) + b2_r[...])
        h = jax.nn.relu(jnp.dot(h, w3_r[...],
                                preferred_element_type=jnp.float32) + b3_r[...])
        g = gu_r[:, :_D] * gi_r[:, :_D]
        p = (jnp.sum(g * wa_r[...], axis=1, keepdims=True)
             + jnp.sum(h * wb_r[...], axis=1, keepdims=True) + bo_r[0, 0])
        out_r[...] = 1.0 / (1.0 + jnp.exp(-p))

    full = lambda a: pl.BlockSpec(a.shape, lambda i: (0,) * a.ndim)
    emb = pl.BlockSpec((bs, 2 * _D), lambda i: (i, 0))
    out = pl.pallas_call(
        body,
        grid=grid,
        in_specs=[emb, emb,
                  full(W1), full(b1r), full(W2), full(b2r),
                  full(W3), full(b3r), full(wa), full(wb), full(bor)],
        out_specs=pl.BlockSpec((bs, 1), lambda i: (i, 0)),
        out_shape=jax.ShapeDtypeStruct((_B, 1), jnp.float32),
    )(gu, gi, W1, b1r, W2, b2r, W3, b3r, wa, wb, bor)
    return out.reshape(_B)


def kernel(user, item, user_gmf_emb, item_gmf_emb, user_mlp_emb, item_mlp_emb,
           W1, b1, W2, b2, W3, b3, Wout, bout):
    U, I = _tc_concat(user_gmf_emb, user_mlp_emb, item_gmf_emb, item_mlp_emb)
    gu, gi = _sc_gather(user, item, U, I)
    return _tc_mlp(gu, gi, W1, b1, W2, b2, W3, b3, Wout, bout)


# speedup vs baseline: 1.8632x; 1.6589x over previous
import functools

import jax
import jax.numpy as jnp
from jax import lax
from jax.experimental import pallas as pl
from jax.experimental.pallas import tpu as pltpu
from jax.experimental.pallas import tpu_sc as plsc

_B = 16384
_D = 64
_NW = 32
_BPW = _B // _NW
_CH = 128
_NPH = _BPW // _CH
_NQ = 8


def _sc_gather(user, item, tug, tig, tum, tim):
    mesh = plsc.VectorSubcoreMesh(core_axis_name="c", subcore_axis_name="s")
    out_t = [jax.ShapeDtypeStruct((_B, 2 * _D), jnp.float32) for _ in range(2)]
    scratch = (
        [pltpu.VMEM((_BPW,), jnp.int32) for _ in range(2)]
        + [pltpu.VMEM((_CH, 2 * _D), jnp.float32) for _ in range(4)]
        + [pltpu.SemaphoreType.DMA for _ in range(2 * _NQ + 2)]
    )

    @functools.partial(pl.kernel, mesh=mesh, out_type=out_t,
                       scratch_types=scratch)
    def body(user_h, item_h, tug_h, tig_h, tum_h, tim_h,
             o_gmf, o_mlp, idxu, idxi, bg0, bg1, bm0, bm1, *sems):
        c = lax.axis_index("c")
        s = lax.axis_index("s")
        base = (s * 2 + c) * _BPW

        pltpu.sync_copy(user_h.at[pl.ds(base, _BPW)], idxu)
        pltpu.sync_copy(item_h.at[pl.ds(base, _BPW)], idxi)

        bgmf = [bg0, bg1]
        bmlp = [bm0, bm1]
        qsems = [sems[:_NQ], sems[_NQ:2 * _NQ]]
        wsems = [sems[2 * _NQ], sems[2 * _NQ + 1]]

        def fire(t):
            off = t * _CH
            k = t % 2
            bg, bm, qs = bgmf[k], bmlp[k], qsems[k]

            def fbody(g, carry):
                vu = idxu[pl.ds(off + g * 16, 16)]
                vi = idxi[pl.ds(off + g * 16, 16)]
                for l in range(16):
                    j = g * 16 + l
                    q = 4 * (l % 2)
                    pltpu.async_copy(tug_h.at[vu[l]],
                                     bg.at[j, pl.ds(0, _D)], qs[q])
                    pltpu.async_copy(tig_h.at[vi[l]],
                                     bg.at[j, pl.ds(_D, _D)], qs[q + 1])
                    pltpu.async_copy(tum_h.at[vu[l]],
                                     bm.at[j, pl.ds(0, _D)], qs[q + 2])
                    pltpu.async_copy(tim_h.at[vi[l]],
                                     bm.at[j, pl.ds(_D, _D)], qs[q + 3])
                return carry
            lax.fori_loop(0, _CH // 16, fbody, 0)

        def drain(t):
            k = t % 2
            for q in range(_NQ):
                pltpu.make_async_copy(
                    o_gmf.at[pl.ds(0, _CH // 4)],
                    bgmf[k].at[pl.ds(0, _CH // 4)], qsems[k][q]).wait()

        fire(0)
        write_handles = [None, None]
        for t in range(_NPH):
            if t + 1 < _NPH:
                fire(t + 1)
            k = t % 2
            drain(t)
            if write_handles[k] is not None:
                for wh in write_handles[k]:
                    wh.wait()
            off = t * _CH
            write_handles[k] = [
                pltpu.async_copy(bgmf[k],
                                 o_gmf.at[pl.ds(base + off, _CH)], wsems[k]),
                pltpu.async_copy(bmlp[k],
                                 o_mlp.at[pl.ds(base + off, _CH)], wsems[k]),
            ]
        for k in range(2):
            if write_handles[k] is not None:
                for wh in write_handles[k]:
                    wh.wait()

    return body(user, item, tug, tig, tum, tim)


def _tc_mlp(gmf2, mlp2, W1, b1, W2, b2, W3, b3, Wout, bout):
    bs = 2048
    grid = (_B // bs,)
    b1r = b1.reshape(1, -1)
    b2r = b2.reshape(1, -1)
    b3r = b3.reshape(1, -1)
    wa = Wout[:_D, 0].reshape(1, _D)
    wb = Wout[_D:, 0].reshape(1, -1)
    bor = bout.reshape(1, 1)

    def body(g_r, m_r, w1_r, b1_r, w2_r, b2_r, w3_r, b3_r,
             wa_r, wb_r, bo_r, out_r):
        h = jnp.dot(m_r[...], w1_r[...], preferred_element_type=jnp.float32)
        h = jax.nn.relu(h + b1_r[...])
        h = jax.nn.relu(jnp.dot(h, w2_r[...],
                                preferred_element_type=jnp.float32) + b2_r[...])
        h = jax.nn.relu(jnp.dot(h, w3_r[...],
                                preferred_element_type=jnp.float32) + b3_r[...])
        g = g_r[:, :_D] * g_r[:, _D:]
        p = (jnp.sum(g * wa_r[...], axis=1, keepdims=True)
             + jnp.sum(h * wb_r[...], axis=1, keepdims=True) + bo_r[0, 0])
        out_r[...] = 1.0 / (1.0 + jnp.exp(-p))

    full = lambda a: pl.BlockSpec(a.shape, lambda i: (0,) * a.ndim)
    emb = pl.BlockSpec((bs, 2 * _D), lambda i: (i, 0))
    out = pl.pallas_call(
        body,
        grid=grid,
        in_specs=[emb, emb,
                  full(W1), full(b1r), full(W2), full(b2r),
                  full(W3), full(b3r), full(wa), full(wb), full(bor)],
        out_specs=pl.BlockSpec((bs, 1), lambda i: (i, 0)),
        out_shape=jax.ShapeDtypeStruct((_B, 1), jnp.float32),
    )(gmf2, mlp2, W1, b1r, W2, b2r, W3, b3r, wa, wb, bor)
    return out.reshape(_B)


def kernel(user, item, user_gmf_emb, item_gmf_emb, user_mlp_emb, item_mlp_emb,
           W1, b1, W2, b2, W3, b3, Wout, bout):
    gmf2, mlp2 = _sc_gather(user, item, user_gmf_emb, item_gmf_emb,
                            user_mlp_emb, item_mlp_emb)
    return _tc_mlp(gmf2, mlp2, W1, b1, W2, b2, W3, b3, Wout, bout)
